# baseline (speedup 1.0000x reference)
import numpy as np
import jax
import jax.numpy as jnp
from jax.experimental import pallas as pl

NP_ = 16
DIMS = 3
H = 32
PK = 4
LW = PK * H
STEPS = 2
TB = 128


def _dense_edge_index():
    idx = np.zeros((NP_, NP_), np.int32)
    mask = np.zeros((NP_ * NP_, 1, 1), np.float32)
    for i in range(NP_):
        for j in range(NP_):
            if i != j:
                idx[i, j] = i * (NP_ - 1) + (j if j < i else j - 1)
                mask[i * NP_ + j] = 1.0
    return idx.reshape(-1), mask


_EDGE_IDX, _EDGE_MASK = _dense_edge_index()


def _erf(x):
    p = 0.3275911
    a1, a2, a3, a4, a5 = (0.254829592, -0.284496736, 1.421413741,
                          -1.453152027, 1.061405429)
    ax = jnp.abs(x)
    t = 1.0 / (1.0 + p * ax)
    poly = ((((a5 * t + a4) * t + a3) * t + a2) * t + a1) * t
    y = 1.0 - poly * jnp.exp(-ax * ax)
    return jnp.sign(x) * y


def _gelu(x):
    return 0.5 * x * (1.0 + _erf(x * 0.7071067811865476))


def _fwd_kernel(nin_ref, xpad_ref, node_Wt, node_b, ee_Wdr_bd, ee_wrr, ee_wr2,
                ee_b1, ee_W2_bd, ee_b2, sumb, v2e_t, euA_bd, euB, euC, eu_b1,
                eu_W2_bd, eu_b2, e2v_bd, nuA, nuB, nu_b1, nu_W2t, nu_b2,
                W_hv, W_he, w_rp, w_r2a, fh_b0, fh_W1t, fh_b1, fh_W2t, fh_b2,
                out_ref):
    f32 = jnp.float32

    def dot32(a, b):
        return jax.lax.dot_general(
            a, b, (((1,), (0,)), ((), ())),
            precision=jax.lax.Precision.HIGHEST,
            preferred_element_type=f32)

    def bf(v):
        return v.astype(jnp.bfloat16).astype(f32)

    def dotb(a, b):
        return jax.lax.dot_general(
            a.astype(jnp.bfloat16), b, (((1,), (0,)), ((), ())),
            preferred_element_type=f32)

    def pack_node(y):
        y3 = y.reshape(TB * PK, PK, H)
        return jnp.concatenate([y3[:, k] for k in range(PK)], axis=1)

    def unpack_node(yp):
        cols = [yp[:, k * H:(k + 1) * H][:, None, :] for k in range(PK)]
        return jnp.concatenate(cols, axis=1).reshape(TB * NP_, H)

    def bc_i(y):
        yt = jnp.concatenate([y] * PK, axis=1)
        return jnp.broadcast_to(
            yt[:, None, :], (TB * NP_, PK, LW)).reshape(TB * NP_ * PK, LW)

    def bc_j(yp):
        y4 = yp.reshape(TB, 1, PK, LW)
        return jnp.broadcast_to(
            y4, (TB, NP_, PK, LW)).reshape(TB * NP_ * PK, LW)

    nin = nin_ref[...]
    x3 = nin[:, :DIMS]
    xpad = xpad_ref[...]

    h_v = dotb(nin, node_Wt[...]) + node_b[...]

    xi = bc_i(xpad)
    xj = bc_j(pack_node(xpad))
    dr = xj - xi
    r2 = dot32(dr * dr, sumb[...])
    rr = jnp.sqrt(r2 + 1e-12)

    h1 = dotb(dr, ee_Wdr_bd[...]) + bf(rr) * ee_wrr[...] \
        + bf(r2) * ee_wr2[...] + ee_b1[...]
    h_e = dotb(_gelu(h1), ee_W2_bd[...]) + ee_b2[...]

    ii = jax.lax.broadcasted_iota(jnp.int32, (1, NP_, PK, LW), 1)
    jh = jax.lax.broadcasted_iota(jnp.int32, (1, NP_, PK, LW), 2)
    lq = jax.lax.broadcasted_iota(jnp.int32, (1, NP_, PK, LW), 3) // H
    nd_mask = (ii != jh * PK + lq).astype(f32)

    for s in range(STEPS):
        v2e = dotb(h_v, v2e_t[s])
        t = dotb(h_e, euA_bd[s]) + bc_i(dotb(v2e, euB[s])) \
            + bc_j(pack_node(dotb(v2e, euC[s]))) + eu_b1[s]
        h_e = dotb(_gelu(t), eu_W2_bd[s]) + eu_b2[s]
        msg = dotb(h_e, e2v_bd[s])
        msgm = msg.reshape(TB, NP_, PK, LW) * nd_mask
        aggp = (jnp.sum(msgm, axis=1) / (NP_ - 1.0)).reshape(TB * PK, LW)
        agg = unpack_node(aggp)
        u = dotb(h_v, nuA[s]) + dotb(agg, nuB[s]) + nu_b1[s]
        h_v = dotb(_gelu(u), nu_W2t[s]) + nu_b2[s]

    xsq = jnp.sum(x3 * x3, axis=1, keepdims=True).reshape(TB, NP_, 1)
    r2_all = jnp.sum(xsq, axis=1)
    x34 = x3.reshape(TB, NP_, DIMS)
    d01 = x34[:, 0, :] - x34[:, 1, :]
    r_pair = jnp.sqrt(jnp.sum(d01 * d01, axis=1, keepdims=True) + 1e-12)

    hvp4 = pack_node(h_v).reshape(TB, PK, LW)
    hv_flat = jnp.concatenate([hvp4[:, k] for k in range(PK)], axis=1)
    hep4 = h_e.reshape(TB, NP_ * PK, LW)
    he_flat = jnp.concatenate(
        [hep4[:, q] for q in range(NP_ * PK)], axis=1)

    acc = dotb(hv_flat, W_hv[...]) + dotb(he_flat, W_he[...]) \
        + bf(r_pair) * w_rp[...] + bf(r2_all) * w_r2a[...] + fh_b0[...]
    h = _gelu(acc)
    h = _gelu(dotb(h, fh_W1t[...]) + fh_b1[...])
    out_ref[...] = dotb(h, fh_W2t[...]) + fh_b2[...]


def _bd(w):
    return jnp.kron(jnp.eye(PK, dtype=w.dtype), w)


def kernel(x, spin, params):
    B = x.shape[0]
    p = params
    f32 = jnp.float32

    node_in = jnp.concatenate(
        [x, spin[..., None].astype(f32)], axis=-1).reshape(B * NP_, DIMS + 1)
    xpad = jnp.pad(x.reshape(B * NP_, DIMS), ((0, 0), (0, H - DIMS)))

    def rnd(w):
        return w.astype(jnp.bfloat16).astype(f32)

    def b16(w):
        return w.astype(jnp.bfloat16)

    eeW1t = rnd(p["ee_W1"].T)
    ee_Wdr = jnp.pad(eeW1t[:DIMS], ((0, H - DIMS), (0, 0)))
    euW1t = rnd(jnp.transpose(p["eu_W1"], (0, 2, 1)))
    nuW1t = rnd(jnp.transpose(p["nu_W1"], (0, 2, 1)))

    fhW0 = rnd(p["fh_W0"])
    W_hv = fhW0[:, :NP_ * H].T
    W240 = fhW0[:, NP_ * H:NP_ * H + 240 * H].T.reshape(240, H, H)
    W_he = (jnp.take(W240, _EDGE_IDX, axis=0) * _EDGE_MASK).reshape(
        NP_ * NP_ * H, H)

    def tile4(v):
        return jnp.tile(v[None], (1, PK))

    sumb = jnp.kron(jnp.eye(PK, dtype=f32), jnp.ones((H, H), f32))

    weights = (
        b16(p["node_W"].T), p["node_b"][None],
        b16(_bd(ee_Wdr)), tile4(eeW1t[DIMS]), tile4(eeW1t[DIMS + 1]),
        tile4(p["ee_b1"]), b16(_bd(p["ee_W2"].T)), tile4(p["ee_b2"]),
        sumb,
        b16(jnp.transpose(p["v2e_W"], (0, 2, 1))),
        b16(jnp.stack([_bd(euW1t[s, :H]) for s in range(STEPS)])),
        b16(euW1t[:, H:2 * H]), b16(euW1t[:, 2 * H:]),
        jnp.stack([tile4(p["eu_b1"][s]) for s in range(STEPS)]),
        b16(jnp.stack([_bd(p["eu_W2"][s].T) for s in range(STEPS)])),
        jnp.stack([tile4(p["eu_b2"][s]) for s in range(STEPS)]),
        b16(jnp.stack([_bd(p["e2v_W"][s].T) for s in range(STEPS)])),
        b16(nuW1t[:, :H]), b16(nuW1t[:, H:]), p["nu_b1"][:, None],
        b16(jnp.transpose(p["nu_W2"], (0, 2, 1))), p["nu_b2"][:, None],
        b16(W_hv), b16(W_he),
        fhW0[:, NP_ * H + 240 * H][None],
        fhW0[:, NP_ * H + 240 * H + 1][None],
        p["fh_b0"][None], b16(p["fh_W1"].T), p["fh_b1"][None],
        b16(p["fh_W2"].T), p["fh_b2"][None],
    )

    def wspec(w):
        return pl.BlockSpec(w.shape, lambda g, nd=w.ndim: (0,) * nd)

    out = pl.pallas_call(
        _fwd_kernel,
        grid=(B // TB,),
        in_specs=[pl.BlockSpec((TB * NP_, DIMS + 1), lambda g: (g, 0)),
                  pl.BlockSpec((TB * NP_, H), lambda g: (g, 0))]
        + [wspec(w) for w in weights],
        out_specs=pl.BlockSpec((TB, 1), lambda g: (g, 0)),
        out_shape=jax.ShapeDtypeStruct((B, 1), f32),
    )(node_in, xpad, *weights)
    return out


# speedup vs baseline: 1.2191x; 1.2191x over previous
import numpy as np
import jax
import jax.numpy as jnp
from jax.experimental import pallas as pl

NP_ = 16
DIMS = 3
H = 32
PK = 4
LW = PK * H
STEPS = 2
TB = 64


def _dense_edge_index():
    idx = np.zeros((NP_, NP_), np.int32)
    mask = np.zeros((NP_ * NP_, 1, 1), np.float32)
    for i in range(NP_):
        for j in range(NP_):
            if i != j:
                idx[i, j] = i * (NP_ - 1) + (j if j < i else j - 1)
                mask[i * NP_ + j] = 1.0
    return idx.reshape(-1), mask


_EDGE_IDX, _EDGE_MASK = _dense_edge_index()


def _erf(x):
    p = 0.3275911
    a1, a2, a3, a4, a5 = (0.254829592, -0.284496736, 1.421413741,
                          -1.453152027, 1.061405429)
    ax = jnp.abs(x)
    t = 1.0 / (1.0 + p * ax)
    poly = ((((a5 * t + a4) * t + a3) * t + a2) * t + a1) * t
    y = 1.0 - poly * jnp.exp(-ax * ax)
    return jnp.sign(x) * y


def _gelu(x):
    return 0.5 * x * (1.0 + _erf(x * 0.7071067811865476))


def _fwd_kernel(nin_ref, xpad_ref, node_Wt, node_b, ee_Wdr_bd, ee_wrr, ee_wr2,
                ee_b1, ee_W2_bd, ee_b2, sumb, v2e_t, euA_bd, euB, euC, eu_b1,
                eu_W2_bd, eu_b2, e2v_bd, nuA, nuB, nu_b1, nu_W2t, nu_b2,
                W_hv, W_he, w_rp, w_r2a, fh_b0, fh_W1t, fh_b1, fh_W2t, fh_b2,
                out_ref):
    f32 = jnp.float32

    def dot32(a, b):
        return jax.lax.dot_general(
            a, b, (((1,), (0,)), ((), ())),
            precision=jax.lax.Precision.HIGHEST,
            preferred_element_type=f32)

    def bf(v):
        return v.astype(jnp.bfloat16).astype(f32)

    def bdot(ab, b):
        return jax.lax.dot_general(
            ab, b, (((1,), (0,)), ((), ())),
            preferred_element_type=f32)

    def dotb(a, b):
        return bdot(a.astype(jnp.bfloat16), b)

    def pack_node(y):
        y3 = y.reshape(TB * PK, PK, H)
        return jnp.concatenate([y3[:, k] for k in range(PK)], axis=1)

    def unpack_node(yp):
        cols = [yp[:, k * H:(k + 1) * H][:, None, :] for k in range(PK)]
        return jnp.concatenate(cols, axis=1).reshape(TB * NP_, H)

    def bc_i(y):
        yt = jnp.concatenate([y] * PK, axis=1)
        return jnp.broadcast_to(
            yt[:, None, :], (TB * NP_, PK, LW)).reshape(TB * NP_ * PK, LW)

    def bc_j(yp):
        y4 = yp.reshape(TB, 1, PK, LW)
        return jnp.broadcast_to(
            y4, (TB, NP_, PK, LW)).reshape(TB * NP_ * PK, LW)

    nin = nin_ref[...]
    x3 = nin[:, :DIMS]
    xpad = xpad_ref[...]

    h_v = dotb(nin, node_Wt[...]) + node_b[...]

    xi = bc_i(xpad)
    xj = bc_j(pack_node(xpad))
    dr = xj - xi
    r2 = dot32(dr * dr, sumb[...])
    rr = jnp.sqrt(r2 + 1e-12)

    h1 = dotb(dr, ee_Wdr_bd[...]) + bf(rr) * ee_wrr[...] \
        + bf(r2) * ee_wr2[...] + ee_b1[...]
    h_e = dotb(_gelu(h1), ee_W2_bd[...]) + ee_b2[...]
    heb = h_e.astype(jnp.bfloat16)

    ii = jax.lax.broadcasted_iota(jnp.int32, (1, NP_, PK, LW), 1)
    jh = jax.lax.broadcasted_iota(jnp.int32, (1, NP_, PK, LW), 2)
    lq = jax.lax.broadcasted_iota(jnp.int32, (1, NP_, PK, LW), 3) // H
    nd_mask = (ii != jh * PK + lq).astype(f32)

    for s in range(STEPS):
        v2eb = dotb(h_v, v2e_t[s]).astype(jnp.bfloat16)
        t = bdot(heb, euA_bd[s]) + bc_i(bdot(v2eb, euB[s])) \
            + bc_j(pack_node(bdot(v2eb, euC[s]))) + eu_b1[s]
        h_e = dotb(_gelu(t), eu_W2_bd[s]) + eu_b2[s]
        heb = h_e.astype(jnp.bfloat16)
        msg = bdot(heb, e2v_bd[s])
        msgm = msg.reshape(TB, NP_, PK, LW) * nd_mask
        aggp = (jnp.sum(msgm, axis=1) / (NP_ - 1.0)).reshape(TB * PK, LW)
        agg = unpack_node(aggp)
        u = dotb(h_v, nuA[s]) + dotb(agg, nuB[s]) + nu_b1[s]
        h_v = dotb(_gelu(u), nu_W2t[s]) + nu_b2[s]

    xsq = jnp.sum(x3 * x3, axis=1, keepdims=True).reshape(TB, NP_, 1)
    r2_all = jnp.sum(xsq, axis=1)
    x34 = x3.reshape(TB, NP_, DIMS)
    d01 = x34[:, 0, :] - x34[:, 1, :]
    r_pair = jnp.sqrt(jnp.sum(d01 * d01, axis=1, keepdims=True) + 1e-12)

    hvp4 = pack_node(h_v.astype(jnp.bfloat16)).reshape(TB, PK, LW)
    hv_flat = jnp.concatenate([hvp4[:, k] for k in range(PK)], axis=1)
    hep4 = heb.reshape(TB, NP_ * PK, LW)
    he_flat = jnp.concatenate(
        [hep4[:, q] for q in range(NP_ * PK)], axis=1)

    acc = bdot(hv_flat, W_hv[...]) + bdot(he_flat, W_he[...]) \
        + bf(r_pair) * w_rp[...] + bf(r2_all) * w_r2a[...] + fh_b0[...]
    h = _gelu(acc)
    h = _gelu(dotb(h, fh_W1t[...]) + fh_b1[...])
    out_ref[...] = dotb(h, fh_W2t[...]) + fh_b2[...]


def _bd(w):
    return jnp.kron(jnp.eye(PK, dtype=w.dtype), w)


def kernel(x, spin, params):
    B = x.shape[0]
    p = params
    f32 = jnp.float32

    node_in = jnp.concatenate(
        [x, spin[..., None].astype(f32)], axis=-1).reshape(B * NP_, DIMS + 1)
    xpad = jnp.pad(x.reshape(B * NP_, DIMS), ((0, 0), (0, H - DIMS)))

    def rnd(w):
        return w.astype(jnp.bfloat16).astype(f32)

    def b16(w):
        return w.astype(jnp.bfloat16)

    eeW1t = rnd(p["ee_W1"].T)
    ee_Wdr = jnp.pad(eeW1t[:DIMS], ((0, H - DIMS), (0, 0)))
    euW1t = rnd(jnp.transpose(p["eu_W1"], (0, 2, 1)))
    nuW1t = rnd(jnp.transpose(p["nu_W1"], (0, 2, 1)))

    fhW0 = rnd(p["fh_W0"])
    W_hv = fhW0[:, :NP_ * H].T
    W240 = fhW0[:, NP_ * H:NP_ * H + 240 * H].T.reshape(240, H, H)
    W_he = (jnp.take(W240, _EDGE_IDX, axis=0) * _EDGE_MASK).reshape(
        NP_ * NP_ * H, H)

    def tile4(v):
        return jnp.tile(v[None], (1, PK))

    sumb = jnp.kron(jnp.eye(PK, dtype=f32), jnp.ones((H, H), f32))

    weights = (
        b16(p["node_W"].T), p["node_b"][None],
        b16(_bd(ee_Wdr)), tile4(eeW1t[DIMS]), tile4(eeW1t[DIMS + 1]),
        tile4(p["ee_b1"]), b16(_bd(p["ee_W2"].T)), tile4(p["ee_b2"]),
        sumb,
        b16(jnp.transpose(p["v2e_W"], (0, 2, 1))),
        b16(jnp.stack([_bd(euW1t[s, :H]) for s in range(STEPS)])),
        b16(euW1t[:, H:2 * H]), b16(euW1t[:, 2 * H:]),
        jnp.stack([tile4(p["eu_b1"][s]) for s in range(STEPS)]),
        b16(jnp.stack([_bd(p["eu_W2"][s].T) for s in range(STEPS)])),
        jnp.stack([tile4(p["eu_b2"][s]) for s in range(STEPS)]),
        b16(jnp.stack([_bd(p["e2v_W"][s].T) for s in range(STEPS)])),
        b16(nuW1t[:, :H]), b16(nuW1t[:, H:]), p["nu_b1"][:, None],
        b16(jnp.transpose(p["nu_W2"], (0, 2, 1))), p["nu_b2"][:, None],
        b16(W_hv), b16(W_he),
        fhW0[:, NP_ * H + 240 * H][None],
        fhW0[:, NP_ * H + 240 * H + 1][None],
        p["fh_b0"][None], b16(p["fh_W1"].T), p["fh_b1"][None],
        b16(p["fh_W2"].T), p["fh_b2"][None],
    )

    def wspec(w):
        return pl.BlockSpec(w.shape, lambda g, nd=w.ndim: (0,) * nd)

    out = pl.pallas_call(
        _fwd_kernel,
        grid=(B // TB,),
        in_specs=[pl.BlockSpec((TB * NP_, DIMS + 1), lambda g: (g, 0)),
                  pl.BlockSpec((TB * NP_, H), lambda g: (g, 0))]
        + [wspec(w) for w in weights],
        out_specs=pl.BlockSpec((TB, 1), lambda g: (g, 0)),
        out_shape=jax.ShapeDtypeStruct((B, 1), f32),
    )(node_in, xpad, *weights)
    return out


# speedup vs baseline: 1.4563x; 1.1946x over previous
import numpy as np
import jax
import jax.numpy as jnp
from jax.experimental import pallas as pl

NP_ = 16
DIMS = 3
H = 32
PK = 4
LW = PK * H
STEPS = 2
TB = 64


def _dense_edge_index():
    idx = np.zeros((NP_, NP_), np.int32)
    mask = np.zeros((NP_ * NP_, 1, 1), np.float32)
    for i in range(NP_):
        for j in range(NP_):
            if i != j:
                idx[i, j] = i * (NP_ - 1) + (j if j < i else j - 1)
                mask[i * NP_ + j] = 1.0
    return idx.reshape(-1), mask


_EDGE_IDX, _EDGE_MASK = _dense_edge_index()


def _erf(x):
    p = 0.3275911
    a1, a2, a3, a4, a5 = (0.254829592, -0.284496736, 1.421413741,
                          -1.453152027, 1.061405429)
    ax = jnp.abs(x)
    t = 1.0 / (1.0 + p * ax)
    poly = ((((a5 * t + a4) * t + a3) * t + a2) * t + a1) * t
    y = 1.0 - poly * jnp.exp(-ax * ax)
    return jnp.sign(x) * y


def _gelu(x):
    return 0.5 * x * (1.0 + _erf(x * 0.7071067811865476))


def _fwd_kernel(nin_ref, xpad_ref, node_Wt, node_b, ee_Wdr_bd, ee_wrr, ee_wr2,
                ee_b1, ee_W2_bd, ee_b2, sumb, v2e_t, euA_bd, euB, euC, eu_b1,
                eu_W2_bd, eu_b2, e2v_bd, nuA, nuB, nu_b1, nu_W2t, nu_b2,
                W_hv3, W_he3, w_rp, w_r2a, fh_b0, fh_W1t, fh_b1, fh_W2t,
                fh_b2, out_ref):
    f32 = jnp.float32
    bf16 = jnp.bfloat16

    def dot32(a, b):
        return jax.lax.dot_general(
            a, b, (((1,), (0,)), ((), ())),
            precision=jax.lax.Precision.HIGHEST,
            preferred_element_type=f32)

    def bf(v):
        return v.astype(bf16).astype(f32)

    def bdot(ab, b):
        return jax.lax.dot_general(
            ab, b, (((1,), (0,)), ((), ())),
            preferred_element_type=f32)

    def dotb(a, b):
        return bdot(a.astype(bf16), b)

    def pack_node(y):
        y6 = y.reshape(PK, PK, TB, H)
        return jnp.concatenate(
            [y6[:, k].reshape(PK * TB, H) for k in range(PK)], axis=1)

    def unpack_node(yp):
        cols = [yp[:, k * H:(k + 1) * H].reshape(PK, 1, TB, H)
                for k in range(PK)]
        return jnp.concatenate(cols, axis=1).reshape(NP_ * TB, H)

    def bc_i(y):
        yt = jnp.concatenate([y] * PK, axis=1)
        y4 = yt.reshape(NP_, 1, TB, LW)
        return jnp.broadcast_to(
            y4, (NP_, PK, TB, LW)).reshape(NP_ * PK * TB, LW)

    def bc_j(yp):
        y3 = yp.reshape(1, PK * TB, LW)
        return jnp.broadcast_to(
            y3, (NP_, PK * TB, LW)).reshape(NP_ * PK * TB, LW)

    nin = nin_ref[...].reshape(NP_ * TB, DIMS + 1)
    x3 = nin[:, :DIMS]
    xpad = xpad_ref[...].reshape(NP_ * TB, H)

    h_v = dotb(nin, node_Wt[...]) + node_b[...]

    dr = bc_j(pack_node(xpad)) - bc_i(xpad)
    r2 = dot32(dr * dr, sumb[...])
    rr = jnp.sqrt(r2 + 1e-12)

    h1 = dotb(dr, ee_Wdr_bd[...]) + bf(rr) * ee_wrr[...] \
        + bf(r2) * ee_wr2[...] + ee_b1[...]
    h_e = dotb(_gelu(h1), ee_W2_bd[...]) + ee_b2[...]
    heb = h_e.astype(bf16)

    ii = jax.lax.broadcasted_iota(jnp.int32, (NP_, PK, 1, LW), 0)
    jh = jax.lax.broadcasted_iota(jnp.int32, (NP_, PK, 1, LW), 1)
    lq = jax.lax.broadcasted_iota(jnp.int32, (NP_, PK, 1, LW), 3) // H
    nd_mask = (ii != jh * PK + lq).astype(f32)

    for s in range(STEPS):
        v2eb = dotb(h_v, v2e_t[s]).astype(bf16)
        t = bdot(heb, euA_bd[s]) + bc_i(bdot(v2eb, euB[s])) \
            + bc_j(pack_node(bdot(v2eb, euC[s]))) + eu_b1[s]
        h_e = dotb(_gelu(t), eu_W2_bd[s]) + eu_b2[s]
        heb = h_e.astype(bf16)
        msg = bdot(heb, e2v_bd[s])
        msgm = msg.reshape(NP_, PK, TB, LW) * nd_mask
        aggp = (jnp.sum(msgm, axis=0) / (NP_ - 1.0)).reshape(PK * TB, LW)
        agg = unpack_node(aggp)
        u = dotb(h_v, nuA[s]) + dotb(agg, nuB[s]) + nu_b1[s]
        h_v = dotb(_gelu(u), nu_W2t[s]) + nu_b2[s]

    x3r = x3.reshape(NP_, TB, DIMS)
    r2_all = jnp.sum(jnp.sum(x3r * x3r, axis=0), axis=1, keepdims=True)
    d01 = x3r[0] - x3r[1]
    r_pair = jnp.sqrt(jnp.sum(d01 * d01, axis=1, keepdims=True) + 1e-12)

    hvb3 = h_v.astype(bf16).reshape(NP_, TB, H)
    heb3 = heb.reshape(NP_ * PK, TB, LW)
    acc = bf(r_pair) * w_rp[...] + bf(r2_all) * w_r2a[...] + fh_b0[...]
    for q in range(NP_):
        acc = acc + bdot(hvb3[q], W_hv3[q])
    for q in range(NP_ * PK):
        acc = acc + bdot(heb3[q], W_he3[q])
    h = _gelu(acc)
    h = _gelu(dotb(h, fh_W1t[...]) + fh_b1[...])
    out_ref[...] = dotb(h, fh_W2t[...]) + fh_b2[...]


def _bd(w):
    return jnp.kron(jnp.eye(PK, dtype=w.dtype), w)


def kernel(x, spin, params):
    B = x.shape[0]
    p = params
    f32 = jnp.float32

    node_in = jnp.transpose(
        jnp.concatenate([x, spin[..., None].astype(f32)], axis=-1), (1, 0, 2))
    xpad = jnp.pad(node_in[..., :DIMS], ((0, 0), (0, 0), (0, H - DIMS)))

    def rnd(w):
        return w.astype(jnp.bfloat16).astype(f32)

    def b16(w):
        return w.astype(jnp.bfloat16)

    eeW1t = rnd(p["ee_W1"].T)
    ee_Wdr = jnp.pad(eeW1t[:DIMS], ((0, H - DIMS), (0, 0)))
    euW1t = rnd(jnp.transpose(p["eu_W1"], (0, 2, 1)))
    nuW1t = rnd(jnp.transpose(p["nu_W1"], (0, 2, 1)))

    fhW0 = rnd(p["fh_W0"])
    W_hv3 = fhW0[:, :NP_ * H].T.reshape(NP_, H, H)
    W240 = fhW0[:, NP_ * H:NP_ * H + 240 * H].T.reshape(240, H, H)
    W_he3 = (jnp.take(W240, _EDGE_IDX, axis=0) * _EDGE_MASK).reshape(
        NP_ * PK, LW, H)

    def tile4(v):
        return jnp.tile(v[None], (1, PK))

    sumb = jnp.kron(jnp.eye(PK, dtype=f32), jnp.ones((H, H), f32))

    weights = (
        b16(p["node_W"].T), p["node_b"][None],
        b16(_bd(ee_Wdr)), tile4(eeW1t[DIMS]), tile4(eeW1t[DIMS + 1]),
        tile4(p["ee_b1"]), b16(_bd(p["ee_W2"].T)), tile4(p["ee_b2"]),
        sumb,
        b16(jnp.transpose(p["v2e_W"], (0, 2, 1))),
        b16(jnp.stack([_bd(euW1t[s, :H]) for s in range(STEPS)])),
        b16(euW1t[:, H:2 * H]), b16(euW1t[:, 2 * H:]),
        jnp.stack([tile4(p["eu_b1"][s]) for s in range(STEPS)]),
        b16(jnp.stack([_bd(p["eu_W2"][s].T) for s in range(STEPS)])),
        jnp.stack([tile4(p["eu_b2"][s]) for s in range(STEPS)]),
        b16(jnp.stack([_bd(p["e2v_W"][s].T) for s in range(STEPS)])),
        b16(nuW1t[:, :H]), b16(nuW1t[:, H:]), p["nu_b1"][:, None],
        b16(jnp.transpose(p["nu_W2"], (0, 2, 1))), p["nu_b2"][:, None],
        b16(W_hv3), b16(W_he3),
        fhW0[:, NP_ * H + 240 * H][None],
        fhW0[:, NP_ * H + 240 * H + 1][None],
        p["fh_b0"][None], b16(p["fh_W1"].T), p["fh_b1"][None],
        b16(p["fh_W2"].T), p["fh_b2"][None],
    )

    def wspec(w):
        return pl.BlockSpec(w.shape, lambda g, nd=w.ndim: (0,) * nd)

    out = pl.pallas_call(
        _fwd_kernel,
        grid=(B // TB,),
        in_specs=[pl.BlockSpec((NP_, TB, DIMS + 1), lambda g: (0, g, 0)),
                  pl.BlockSpec((NP_, TB, H), lambda g: (0, g, 0))]
        + [wspec(w) for w in weights],
        out_specs=pl.BlockSpec((TB, 1), lambda g: (g, 0)),
        out_shape=jax.ShapeDtypeStruct((B, 1), f32),
    )(node_in, xpad, *weights)
    return out
